# baseline (speedup 1.0000x reference)
import functools

import jax
import jax.numpy as jnp
from jax import lax
from jax.experimental import pallas as pl
from jax.experimental.pallas import tpu as pltpu
from jax.experimental.pallas import tpu_sc as plsc

_NC = 2
_NS = 16
_L = 16
_D = 128
_W = _D + _L
_CH = 80


def _tc_xw_body(x_ref, w_ref, r_ref, ob_ref, or_ref):
    xb = x_ref[...]
    k = w_ref.shape[0]
    dk = [jnp.dot(xb, w_ref[i], preferred_element_type=jnp.float32
                  ).astype(jnp.bfloat16) for i in range(k)]
    for i in range(k - 1):
        ob_ref[i, :, :_D] = dk[i]
        ob_ref[i, :, _D:] = dk[i + 1]
    or_ref[...] = jnp.dot(xb, r_ref[...], preferred_element_type=jnp.float32)


def _tc_xw(x, weight, root, bn=1024):
    n, d = x.shape
    k = weight.shape[0]
    return pl.pallas_call(
        _tc_xw_body,
        grid=(n // bn,),
        in_specs=[
            pl.BlockSpec((bn, d), lambda i: (i, 0)),
            pl.BlockSpec((k, d, d), lambda i: (0, 0, 0)),
            pl.BlockSpec((d, d), lambda i: (0, 0)),
        ],
        out_specs=[
            pl.BlockSpec((k - 1, bn, 2 * d), lambda i: (0, i, 0)),
            pl.BlockSpec((bn, d), lambda i: (i, 0)),
        ],
        out_shape=[
            jax.ShapeDtypeStruct((k - 1, n, 2 * d), jnp.bfloat16),
            jax.ShapeDtypeStruct((n, d), jnp.float32),
        ],
    )(x, weight, root)


def _node_update(agg_ref, hist_ref, xr_ref, b_ref):
    ssum = agg_ref[0] + agg_ref[1]
    cnt = (hist_ref[0] + hist_ref[1])[:, None]
    h = ssum / jnp.maximum(cnt, 1.0) + xr_ref[...] + b_ref[...]
    return jnp.where(h > 0.0, h, jnp.exp(jnp.minimum(h, 0.0)) - 1.0)


def _tc_update_xw_body(agg_ref, hist_ref, xr_ref, b_ref, w_ref, r_ref,
                       ob_ref, or_ref):
    h = _node_update(agg_ref, hist_ref, xr_ref, b_ref)
    k = w_ref.shape[0]
    dk = [jnp.dot(h, w_ref[i], preferred_element_type=jnp.float32
                  ).astype(jnp.bfloat16) for i in range(k)]
    for i in range(k - 1):
        ob_ref[i, :, :_D] = dk[i]
        ob_ref[i, :, _D:] = dk[i + 1]
    or_ref[...] = jnp.dot(h, r_ref[...], preferred_element_type=jnp.float32)


def _tc_update_xw(agg, hist, xroot, bias, weight, root, bn=1024):
    n = xroot.shape[0]
    d = _D
    k = weight.shape[0]
    return pl.pallas_call(
        _tc_update_xw_body,
        grid=(n // bn,),
        in_specs=[
            pl.BlockSpec((_NC, bn, d), lambda i: (0, i, 0)),
            pl.BlockSpec((_NC, bn), lambda i: (0, i)),
            pl.BlockSpec((bn, d), lambda i: (i, 0)),
            pl.BlockSpec((1, d), lambda i: (0, 0)),
            pl.BlockSpec((k, d, d), lambda i: (0, 0, 0)),
            pl.BlockSpec((d, d), lambda i: (0, 0)),
        ],
        out_specs=[
            pl.BlockSpec((k - 1, bn, 2 * d), lambda i: (0, i, 0)),
            pl.BlockSpec((bn, d), lambda i: (i, 0)),
        ],
        out_shape=[
            jax.ShapeDtypeStruct((k - 1, n, 2 * d), jnp.bfloat16),
            jax.ShapeDtypeStruct((n, d), jnp.float32),
        ],
    )(agg, hist, xroot, bias.reshape(1, d), weight, root)


def _tc_final_body(agg_ref, hist_ref, xr_ref, b_ref, mw_ref, mb_ref, o_ref):
    h = _node_update(agg_ref, hist_ref, xr_ref, b_ref)
    o = jnp.dot(h, mw_ref[...], preferred_element_type=jnp.float32) + mb_ref[...]
    o_ref[...] = jnp.maximum(o, 0.0)


def _tc_final(agg, hist, xroot, bias, mlp_w, mlp_b, bn=1024):
    n = xroot.shape[0]
    d = _D
    c = mlp_w.shape[1]
    return pl.pallas_call(
        _tc_final_body,
        grid=(n // bn,),
        in_specs=[
            pl.BlockSpec((_NC, bn, d), lambda i: (0, i, 0)),
            pl.BlockSpec((_NC, bn), lambda i: (0, i)),
            pl.BlockSpec((bn, d), lambda i: (i, 0)),
            pl.BlockSpec((1, d), lambda i: (0, 0)),
            pl.BlockSpec((d, c), lambda i: (0, 0)),
            pl.BlockSpec((1, c), lambda i: (0, 0)),
        ],
        out_specs=pl.BlockSpec((bn, c), lambda i: (i, 0)),
        out_shape=jax.ShapeDtypeStruct((n, c), jnp.float32),
    )(agg, hist, xroot, bias.reshape(1, d), mlp_w, mlp_b.reshape(1, c))


def _sc_cnt(dst2, n_pad):
    epw = dst2.shape[1]
    ch = n_pad // _D
    mesh = plsc.VectorSubcoreMesh(core_axis_name="c", subcore_axis_name="s")

    @functools.partial(
        pl.kernel,
        out_type=jax.ShapeDtypeStruct((_NC, ch, _D), jnp.float32),
        mesh=mesh,
        compiler_params=pltpu.CompilerParams(needs_layout_passes=False),
        scratch_types=[
            pltpu.VMEM((epw,), jnp.int32),
            pltpu.VMEM((ch, _D), jnp.float32),
            pltpu.VMEM((ch,), jnp.int32),
            pltpu.VMEM((16, _D), jnp.float32),
            pltpu.VMEM_SHARED((ch, _D), jnp.float32),
        ],
    )
    def k(dst_hbm, cnt_hbm, dstv, hist, idc, zbuf, cnt_sh):
        c = lax.axis_index("c")
        s = lax.axis_index("s")
        wid = c * _NS + s
        zeros16 = jnp.zeros((_L,), jnp.float32)
        ones16 = jnp.ones((_L,), jnp.float32)

        pltpu.sync_copy(dst_hbm.at[wid], dstv)

        def zhist(r, carry):
            for j in range(_D // _L):
                hist[r, pl.ds(j * _L, _L)] = zeros16
            return carry
        lax.fori_loop(0, ch, zhist, 0)
        for g in range(ch // _L):
            idc[pl.ds(g * _L, _L)] = lax.iota(jnp.int32, _L) + g * _L

        @pl.when(s == 0)
        def _():
            for j in range(_D // _L):
                for r in range(16):
                    zbuf[r, pl.ds(j * _L, _L)] = zeros16
            for i in range(ch // 16):
                pltpu.sync_copy(zbuf, cnt_sh.at[pl.ds(i * 16, 16)])

        def group(g, carry):
            dd = dstv[pl.ds(g * _L, _L)]
            plsc.addupdate_scatter(
                hist, [lax.shift_right_logical(dd, 7),
                       lax.bitwise_and(dd, _D - 1)], ones16)
            return carry
        plsc.subcore_barrier()
        lax.fori_loop(0, epw // _L, group, 0)

        pltpu.sync_copy(hist, cnt_sh.at[idc], add=True)
        plsc.subcore_barrier()

        @pl.when(s == 0)
        def _():
            pltpu.sync_copy(cnt_sh, cnt_hbm.at[c])

    return k(dst2)


def _sc_agg(xw_flat, e3, n_pad, k_basis):
    nch = e3.shape[1]
    rps = n_pad // _NS
    km1 = k_basis - 1
    mesh = plsc.VectorSubcoreMesh(core_axis_name="c", subcore_axis_name="s")

    scratch = [
        pltpu.VMEM_SHARED((n_pad, _D), jnp.float32),
        pltpu.VMEM((16, _D), jnp.float32),
    ]
    for _ in range(2):
        scratch += [
            pltpu.VMEM((3, _CH), jnp.int32),
            pltpu.VMEM((_CH,), jnp.int32),
            pltpu.VMEM((_CH,), jnp.int32),
            pltpu.VMEM((_CH,), jnp.float32),
            pltpu.VMEM((_CH, _D), jnp.int32),
            pltpu.VMEM((_CH, _D), jnp.float32),
            pltpu.SemaphoreType.DMA,
            pltpu.SemaphoreType.DMA,
            pltpu.SemaphoreType.DMA,
        ]

    @functools.partial(
        pl.kernel,
        out_type=jax.ShapeDtypeStruct((_NC, n_pad, _D), jnp.float32),
        mesh=mesh,
        compiler_params=pltpu.CompilerParams(needs_layout_passes=False),
        scratch_types=scratch,
    )
    def k(xw_hbm, e3_hbm, out_hbm, agg_sh, zbuf, *bufs):
        eb = (bufs[0], bufs[9])
        i0v = (bufs[1], bufs[10])
        d2v = (bufs[2], bufs[11])
        fv = (bufs[3], bufs[12])
        rows0 = (bufs[4], bufs[13])
        msg = (bufs[5], bufs[14])
        sem_e = (bufs[6], bufs[15])
        sem_g = (bufs[7], bufs[16])
        sem_s = (bufs[8], bufs[17])
        c = lax.axis_index("c")
        s = lax.axis_index("s")
        wid = c * _NS + s
        zeros16 = jnp.zeros((_L,), jnp.float32)

        for r in range(16):
            for j in range(_D // _L):
                zbuf[r, pl.ds(j * _L, _L)] = zeros16

        def zcopy(i, carry):
            pltpu.sync_copy(zbuf, agg_sh.at[pl.ds(s * rps + i * 16, 16)])
            return carry
        lax.fori_loop(0, rps // 16, zcopy, 0)

        def calc_idx(b, t):
            for g in range(_CH // _L):
                sl = pl.ds(g * _L, _L)
                v = plsc.bitcast(eb[b][2, sl], jnp.float32) * float(km1)
                bb = jnp.minimum(v.astype(jnp.int32), km1 - 1)
                i0v[b][sl] = bb * n_pad + eb[b][0, sl]
                d2v[b][sl] = eb[b][1, sl]
                fv[b][sl] = v - bb.astype(jnp.float32)

        def gather(b, t):
            pltpu.async_copy(xw_hbm.at[i0v[b]], rows0[b], sem_g[b])

        def wait_gather(b):
            pltpu.make_async_copy(xw_hbm.at[pl.ds(0, _CH)], rows0[b], sem_g[b]).wait()

        def wait_scatter(b):
            pltpu.make_async_copy(out_hbm.at[0, pl.ds(0, _CH)], msg[b],
                                  sem_s[b]).wait()

        plsc.subcore_barrier()

        pltpu.sync_copy(e3_hbm.at[wid, 0], eb[0])
        calc_idx(0, 0)
        gather(0, 0)
        pltpu.async_copy(e3_hbm.at[wid, 1], eb[1], sem_e[1])

        def blend_scatter(b):
            hmask = jnp.full((_L,), -65536, jnp.int32)

            def group(gg, gcarry):
                base = gg * _L
                for ee in range(_L):
                    e = base + ee
                    fvec = plsc.load_gather(
                        fv[b], [jnp.full((_L,), e, jnp.int32)])
                    for jj in range(_D // (2 * _L)):
                        v0 = rows0[b][e, pl.ds(jj * _L, _L)]
                        v1 = rows0[b][e, pl.ds(_D // 2 + jj * _L, _L)]
                        lo0 = plsc.bitcast(lax.shift_left(v0, 16), jnp.float32)
                        lo1 = plsc.bitcast(lax.shift_left(v1, 16), jnp.float32)
                        hi0 = plsc.bitcast(lax.bitwise_and(v0, hmask), jnp.float32)
                        hi1 = plsc.bitcast(lax.bitwise_and(v1, hmask), jnp.float32)
                        msg[b][e, pl.ds(2 * jj * _L, _L)] = lo0 + fvec * (lo1 - lo0)
                        msg[b][e, pl.ds((2 * jj + 1) * _L, _L)] = \
                            hi0 + fvec * (hi1 - hi0)
                return gcarry
            lax.fori_loop(0, _CH // _L, group, 0)
            pltpu.async_copy(msg[b], agg_sh.at[d2v[b]], sem_s[b], add=True)

        def pair(j, carry):
            for b in (0, 1):
                t = 2 * j + b
                nb = 1 - b

                @pl.when(t >= 1)
                def _():
                    wait_scatter(nb)
                pltpu.make_async_copy(
                    e3_hbm.at[wid, 0], eb[nb], sem_e[nb]).wait()
                calc_idx(nb, t + 1)
                gather(nb, t + 1)

                @pl.when(t < nch - 2)
                def _():
                    pltpu.async_copy(e3_hbm.at[wid, t + 2], eb[b], sem_e[b])

                wait_gather(b)
                blend_scatter(b)
            return carry
        lax.fori_loop(0, nch // 2, pair, 0)

        wait_gather(0)
        blend_scatter(0)
        wait_scatter(1)
        wait_scatter(0)

        plsc.subcore_barrier()

        def ocopy(i, carry):
            r0 = s * rps + i * 64
            pltpu.sync_copy(agg_sh.at[pl.ds(r0, 64)],
                            out_hbm.at[c, pl.ds(r0, 64)])
            return carry
        lax.fori_loop(0, rps // 64, ocopy, 0)

    return k(xw_flat, e3)


def kernel(x, edge_index, edge_attr, weight1, root1, bias1,
           weight2, root2, bias2, mlp_w, mlp_b):
    n, d = x.shape
    e = edge_index.shape[1]
    nw = _NC * _NS
    epw = e // nw
    n_pad = 2048 * -(-n // 2048)
    x_pad = jnp.pad(x, ((0, n_pad - n), (0, 0)))
    u_bits = lax.bitcast_convert_type(edge_attr[:, 0], jnp.int32)
    e3 = jnp.stack([edge_index[0], edge_index[1], u_bits], axis=0)
    e3 = e3.reshape(3, nw, epw // _CH, _CH).transpose(1, 2, 0, 3)
    dst2 = edge_index[1].reshape(nw, epw)

    import numpy as np
    blk = np.arange(d).reshape(d // 32, 2, 16)
    perm = np.concatenate(
        [np.stack([32 * b + 2 * np.arange(16), 32 * b + 2 * np.arange(16) + 1])
         for b in range(d // 32)]).reshape(d // 32, 2, 16)
    perm = perm.reshape(d // 32 * 2, 16).reshape(-1)
    del blk
    root1p = root1[:, perm]
    bias1p = bias1[perm]
    weight2p = weight2[:, perm, :]
    root2p = root2[perm][:, perm]
    bias2p = bias2[perm]
    mlp_wp = mlp_w[perm, :]

    cnt = _sc_cnt(dst2, n_pad)
    cnt2 = cnt.reshape(_NC, n_pad)
    xwb1, xroot1 = _tc_xw(x_pad, weight1, root1p)
    k1 = weight1.shape[0]
    k2 = weight2.shape[0]
    xwi1 = lax.bitcast_convert_type(
        xwb1.reshape((k1 - 1) * n_pad, d, 2), jnp.int32)
    agg1 = _sc_agg(xwi1, e3, n_pad, k1)
    xwb2, hroot2 = _tc_update_xw(agg1, cnt2, xroot1, bias1p, weight2p, root2p)
    xwi2 = lax.bitcast_convert_type(
        xwb2.reshape((k2 - 1) * n_pad, d, 2), jnp.int32)
    agg2 = _sc_agg(xwi2, e3, n_pad, k2)
    return _tc_final(agg2, cnt2, hroot2, bias2p, mlp_wp, mlp_b)[:n]


# speedup vs baseline: 1.4077x; 1.4077x over previous
import functools

import jax
import jax.numpy as jnp
from jax import lax
from jax.experimental import pallas as pl
from jax.experimental.pallas import tpu as pltpu
from jax.experimental.pallas import tpu_sc as plsc

_NC = 2
_NS = 16
_L = 16
_D = 128
_W = _D + _L
_CH = 80


def _tc_xw_body(x_ref, w_ref, r_ref, o_ref):
    xb = x_ref[...]
    k = w_ref.shape[0]
    for i in range(k):
        o_ref[i] = jnp.dot(xb, w_ref[i], preferred_element_type=jnp.float32)
    o_ref[k] = jnp.dot(xb, r_ref[...], preferred_element_type=jnp.float32)


def _tc_xw(x, weight, root, bn=1024):
    n, d = x.shape
    k = weight.shape[0]
    return pl.pallas_call(
        _tc_xw_body,
        grid=(n // bn,),
        in_specs=[
            pl.BlockSpec((bn, d), lambda i: (i, 0)),
            pl.BlockSpec((k, d, d), lambda i: (0, 0, 0)),
            pl.BlockSpec((d, d), lambda i: (0, 0)),
        ],
        out_specs=pl.BlockSpec((k + 1, bn, d), lambda i: (0, i, 0)),
        out_shape=jax.ShapeDtypeStruct((k + 1, n, d), jnp.float32),
    )(x, weight, root)


def _node_update(agg_ref, hist_ref, xr_ref, b_ref):
    ssum = agg_ref[0] + agg_ref[1]
    cnt = (hist_ref[0] + hist_ref[1])[:, None]
    h = ssum / jnp.maximum(cnt, 1.0) + xr_ref[0] + b_ref[...]
    return jnp.where(h > 0.0, h, jnp.exp(jnp.minimum(h, 0.0)) - 1.0)


def _tc_update_xw_body(agg_ref, hist_ref, xr_ref, b_ref, w_ref, r_ref, o_ref):
    h = _node_update(agg_ref, hist_ref, xr_ref, b_ref)
    k = w_ref.shape[0]
    for i in range(k):
        o_ref[i] = jnp.dot(h, w_ref[i], preferred_element_type=jnp.float32)
    o_ref[k] = jnp.dot(h, r_ref[...], preferred_element_type=jnp.float32)


def _tc_update_xw(agg, hist, xw_prev, bias, weight, root, k_prev, bn=1024):
    n = xw_prev.shape[1]
    d = _D
    k = weight.shape[0]
    return pl.pallas_call(
        _tc_update_xw_body,
        grid=(n // bn,),
        in_specs=[
            pl.BlockSpec((_NC, bn, d), lambda i: (0, i, 0)),
            pl.BlockSpec((_NC, bn), lambda i: (0, i)),
            pl.BlockSpec((1, bn, d), lambda i: (k_prev, i, 0)),
            pl.BlockSpec((1, d), lambda i: (0, 0)),
            pl.BlockSpec((k, d, d), lambda i: (0, 0, 0)),
            pl.BlockSpec((d, d), lambda i: (0, 0)),
        ],
        out_specs=pl.BlockSpec((k + 1, bn, d), lambda i: (0, i, 0)),
        out_shape=jax.ShapeDtypeStruct((k + 1, n, d), jnp.float32),
    )(agg, hist, xw_prev, bias.reshape(1, d), weight, root)


def _tc_final_body(agg_ref, hist_ref, xr_ref, b_ref, mw_ref, mb_ref, o_ref):
    h = _node_update(agg_ref, hist_ref, xr_ref, b_ref)
    o = jnp.dot(h, mw_ref[...], preferred_element_type=jnp.float32) + mb_ref[...]
    o_ref[...] = jnp.maximum(o, 0.0)


def _tc_final(agg, hist, xw_prev, bias, mlp_w, mlp_b, k_prev, bn=1024):
    n = xw_prev.shape[1]
    d = _D
    c = mlp_w.shape[1]
    return pl.pallas_call(
        _tc_final_body,
        grid=(n // bn,),
        in_specs=[
            pl.BlockSpec((_NC, bn, d), lambda i: (0, i, 0)),
            pl.BlockSpec((_NC, bn), lambda i: (0, i)),
            pl.BlockSpec((1, bn, d), lambda i: (k_prev, i, 0)),
            pl.BlockSpec((1, d), lambda i: (0, 0)),
            pl.BlockSpec((d, c), lambda i: (0, 0)),
            pl.BlockSpec((1, c), lambda i: (0, 0)),
        ],
        out_specs=pl.BlockSpec((bn, c), lambda i: (i, 0)),
        out_shape=jax.ShapeDtypeStruct((n, c), jnp.float32),
    )(agg, hist, xw_prev, bias.reshape(1, d), mlp_w, mlp_b.reshape(1, c))


def _sc_cnt(dst2, n_pad):
    epw = dst2.shape[1]
    ch = n_pad // _D
    mesh = plsc.VectorSubcoreMesh(core_axis_name="c", subcore_axis_name="s")

    @functools.partial(
        pl.kernel,
        out_type=jax.ShapeDtypeStruct((_NC, ch, _D), jnp.float32),
        mesh=mesh,
        compiler_params=pltpu.CompilerParams(needs_layout_passes=False),
        scratch_types=[
            pltpu.VMEM((epw,), jnp.int32),
            pltpu.VMEM((ch, _D), jnp.float32),
            pltpu.VMEM((ch,), jnp.int32),
            pltpu.VMEM((16, _D), jnp.float32),
            pltpu.VMEM_SHARED((ch, _D), jnp.float32),
        ],
    )
    def k(dst_hbm, cnt_hbm, dstv, hist, idc, zbuf, cnt_sh):
        c = lax.axis_index("c")
        s = lax.axis_index("s")
        wid = c * _NS + s
        zeros16 = jnp.zeros((_L,), jnp.float32)
        ones16 = jnp.ones((_L,), jnp.float32)

        pltpu.sync_copy(dst_hbm.at[wid], dstv)

        def zhist(r, carry):
            for j in range(_D // _L):
                hist[r, pl.ds(j * _L, _L)] = zeros16
            return carry
        lax.fori_loop(0, ch, zhist, 0)
        for g in range(ch // _L):
            idc[pl.ds(g * _L, _L)] = lax.iota(jnp.int32, _L) + g * _L

        @pl.when(s == 0)
        def _():
            for j in range(_D // _L):
                for r in range(16):
                    zbuf[r, pl.ds(j * _L, _L)] = zeros16
            for i in range(ch // 16):
                pltpu.sync_copy(zbuf, cnt_sh.at[pl.ds(i * 16, 16)])

        def group(g, carry):
            dd = dstv[pl.ds(g * _L, _L)]
            plsc.addupdate_scatter(
                hist, [lax.shift_right_logical(dd, 7),
                       lax.bitwise_and(dd, _D - 1)], ones16)
            return carry
        plsc.subcore_barrier()
        lax.fori_loop(0, epw // _L, group, 0)

        pltpu.sync_copy(hist, cnt_sh.at[idc], add=True)
        plsc.subcore_barrier()

        @pl.when(s == 0)
        def _():
            pltpu.sync_copy(cnt_sh, cnt_hbm.at[c])

    return k(dst2)


def _sc_agg(xw_flat, e3, n_pad, k_basis):
    nch = e3.shape[1]
    rps = n_pad // _NS
    km1 = k_basis - 1
    mesh = plsc.VectorSubcoreMesh(core_axis_name="c", subcore_axis_name="s")

    scratch = [
        pltpu.VMEM_SHARED((n_pad, _D), jnp.float32),
        pltpu.VMEM((16, _D), jnp.float32),
    ]
    for _ in range(2):
        scratch += [
            pltpu.VMEM((3, _CH), jnp.int32),
            pltpu.VMEM((_CH,), jnp.int32),
            pltpu.VMEM((_CH,), jnp.int32),
            pltpu.VMEM((_CH,), jnp.int32),
            pltpu.VMEM((_CH,), jnp.float32),
            pltpu.VMEM((_CH, _D), jnp.float32),
            pltpu.VMEM((_CH, _D), jnp.float32),
            pltpu.SemaphoreType.DMA,
            pltpu.SemaphoreType.DMA,
            pltpu.SemaphoreType.DMA,
        ]

    @functools.partial(
        pl.kernel,
        out_type=jax.ShapeDtypeStruct((_NC, n_pad, _D), jnp.float32),
        mesh=mesh,
        compiler_params=pltpu.CompilerParams(needs_layout_passes=False),
        scratch_types=scratch,
    )
    def k(xw_hbm, e3_hbm, out_hbm, agg_sh, zbuf, *bufs):
        eb = (bufs[0], bufs[10])
        i0v = (bufs[1], bufs[11])
        i1v = (bufs[2], bufs[12])
        d2v = (bufs[3], bufs[13])
        fv = (bufs[4], bufs[14])
        rows0 = (bufs[5], bufs[15])
        rows1 = (bufs[6], bufs[16])
        sem_e = (bufs[7], bufs[17])
        sem_g = (bufs[8], bufs[18])
        sem_s = (bufs[9], bufs[19])
        c = lax.axis_index("c")
        s = lax.axis_index("s")
        wid = c * _NS + s
        zeros16 = jnp.zeros((_L,), jnp.float32)

        for r in range(16):
            for j in range(_D // _L):
                zbuf[r, pl.ds(j * _L, _L)] = zeros16

        def zcopy(i, carry):
            pltpu.sync_copy(zbuf, agg_sh.at[pl.ds(s * rps + i * 16, 16)])
            return carry
        lax.fori_loop(0, rps // 16, zcopy, 0)

        def calc_idx(b, t):
            for g in range(_CH // _L):
                sl = pl.ds(g * _L, _L)
                v = plsc.bitcast(eb[b][2, sl], jnp.float32) * float(km1)
                bb = jnp.minimum(v.astype(jnp.int32), km1 - 1)
                i0 = bb * n_pad + eb[b][0, sl]
                i0v[b][sl] = i0
                i1v[b][sl] = i0 + n_pad
                d2v[b][sl] = eb[b][1, sl]
                fv[b][sl] = v - bb.astype(jnp.float32)

        def gather(b, t):
            pltpu.async_copy(xw_hbm.at[i0v[b]], rows0[b], sem_g[b])
            pltpu.async_copy(xw_hbm.at[i1v[b]], rows1[b], sem_g[b])

        def wait_gather(b):
            pltpu.make_async_copy(xw_hbm.at[pl.ds(0, _CH)], rows0[b], sem_g[b]).wait()
            pltpu.make_async_copy(xw_hbm.at[pl.ds(0, _CH)], rows1[b], sem_g[b]).wait()

        def wait_scatter(b):
            pltpu.make_async_copy(xw_hbm.at[pl.ds(0, _CH)], rows0[b], sem_s[b]).wait()

        plsc.subcore_barrier()

        pltpu.sync_copy(e3_hbm.at[wid, 0], eb[0])
        calc_idx(0, 0)
        gather(0, 0)
        pltpu.async_copy(e3_hbm.at[wid, 1], eb[1], sem_e[1])

        def blend_scatter(b):
            def group(gg, gcarry):
                base = gg * _L
                for ee in range(_L):
                    e = base + ee
                    fvec = plsc.load_gather(
                        fv[b], [jnp.full((_L,), e, jnp.int32)])
                    for jj in range(_D // _L):
                        sl = pl.ds(jj * _L, _L)
                        r0 = rows0[b][e, sl]
                        r1 = rows1[b][e, sl]
                        rows0[b][e, sl] = r0 + fvec * (r1 - r0)
                return gcarry
            lax.fori_loop(0, _CH // _L, group, 0)
            pltpu.async_copy(rows0[b], agg_sh.at[d2v[b]], sem_s[b], add=True)

        def pair(j, carry):
            for b in (0, 1):
                t = 2 * j + b
                nb = 1 - b

                @pl.when(t >= 1)
                def _():
                    wait_scatter(nb)
                pltpu.make_async_copy(
                    e3_hbm.at[wid, 0], eb[nb], sem_e[nb]).wait()
                calc_idx(nb, t + 1)
                gather(nb, t + 1)

                @pl.when(t < nch - 2)
                def _():
                    pltpu.async_copy(e3_hbm.at[wid, t + 2], eb[b], sem_e[b])

                wait_gather(b)
                blend_scatter(b)
            return carry
        lax.fori_loop(0, nch // 2, pair, 0)

        wait_gather(0)
        blend_scatter(0)
        wait_scatter(1)
        wait_scatter(0)

        plsc.subcore_barrier()

        def ocopy(i, carry):
            r0 = s * rps + i * 64
            pltpu.sync_copy(agg_sh.at[pl.ds(r0, 64)],
                            out_hbm.at[c, pl.ds(r0, 64)])
            return carry
        lax.fori_loop(0, rps // 64, ocopy, 0)

    return k(xw_flat, e3)


def kernel(x, edge_index, edge_attr, weight1, root1, bias1,
           weight2, root2, bias2, mlp_w, mlp_b):
    n, d = x.shape
    e = edge_index.shape[1]
    k1 = weight1.shape[0]
    k2 = weight2.shape[0]
    nw = _NC * _NS
    epw = e // nw
    n_pad = 2048 * -(-n // 2048)
    x_pad = jnp.pad(x, ((0, n_pad - n), (0, 0)))
    u_bits = lax.bitcast_convert_type(edge_attr[:, 0], jnp.int32)
    e3 = jnp.stack([edge_index[0], edge_index[1], u_bits], axis=0)
    e3 = e3.reshape(3, nw, epw // _CH, _CH).transpose(1, 2, 0, 3)
    dst2 = edge_index[1].reshape(nw, epw)

    cnt = _sc_cnt(dst2, n_pad)
    cnt2 = cnt.reshape(_NC, n_pad)
    xw1 = _tc_xw(x_pad, weight1, root1)
    agg1 = _sc_agg(xw1.reshape((k1 + 1) * n_pad, d), e3, n_pad, k1)
    xw2 = _tc_update_xw(agg1, cnt2, xw1, bias1, weight2, root2, k1)
    agg2 = _sc_agg(xw2.reshape((k2 + 1) * n_pad, d), e3, n_pad, k2)
    return _tc_final(agg2, cnt2, xw2, bias2, mlp_w, mlp_b, k2)[:n]
